# baseline (speedup 1.0000x reference)
import jax
import jax.numpy as jnp
from jax import lax
from jax.experimental import pallas as pl
from jax.experimental.pallas import tpu as pltpu
from jax.experimental.pallas import tpu_sc as plsc

_NC = 2
_NS = 16
_NW = _NC * _NS
_CHUNK = 128
_D = 64
_DT = _D // 8
_NBUF = 4
_AHEAD = 2


def _body(x_hbm, table_hbm, out_hbm, idx_v, rows_v, tile_v, gsem, ssem):
    nchunk = x_hbm.shape[1]
    n_s = out_hbm.shape[0]
    wid = lax.axis_index("s") * _NC + lax.axis_index("c")
    pltpu.sync_copy(x_hbm.at[wid], idx_v)
    iota = lax.broadcasted_iota(jnp.int32, (16,), 0)

    def g_desc(j, b):
        return pltpu.make_async_copy(
            table_hbm.at[idx_v.at[j]], rows_v.at[b], gsem)

    def s_desc(j, b):
        t = j // n_s
        s = j - t * n_s
        return pltpu.make_async_copy(
            tile_v.at[b], out_hbm.at[s, :, wid * (nchunk // n_s) + t], ssem)

    for j in range(_AHEAD):
        g_desc(j, j % _NBUF).start()

    def transpose_chunk(b):
        rows_b = rows_v.at[b]
        tile_b = tile_v.at[b]

        @plsc.parallel_loop(0, _D, unroll=4)
        def _(d):
            dt = d // 8
            ds = d - dt * 8
            col = jnp.full((16,), d, jnp.int32)
            vs = [plsc.load_gather(rows_b, [iota + 16 * k, col])
                  for k in range(8)]
            for k in range(8):
                tile_b[dt, ds, pl.ds(16 * k, 16)] = vs[k]

    def group(g, carry):
        for b in range(_NBUF):
            j = g * _NBUF + b

            @pl.when(j + _AHEAD < nchunk)
            def _():
                g_desc(j + _AHEAD, (b + _AHEAD) % _NBUF).start()

            g_desc(j, b).wait()

            @pl.when(g >= 1)
            def _():
                s_desc(j - _NBUF, b).wait()

            transpose_chunk(b)
            s_desc(j, b).start()
        return carry

    lax.fori_loop(0, nchunk // _NBUF, group, 0)
    for j in range(nchunk - _NBUF, nchunk):
        s_desc(j, j % _NBUF).wait()


def kernel(x, table):
    bsz, n_s = x.shape
    nbt = bsz // _CHUNK
    tpw = nbt // _NW
    nchunk = tpw * n_s
    xt = (x.reshape(_NW, tpw, _CHUNK, n_s)
          .transpose(0, 1, 3, 2)
          .reshape(_NW, nchunk, _CHUNK)
          .astype(jnp.int32))
    mesh = plsc.VectorSubcoreMesh(core_axis_name="c", subcore_axis_name="s")
    out5 = pl.kernel(
        _body,
        out_type=jax.ShapeDtypeStruct((n_s, _DT, nbt, 8, _CHUNK), jnp.float32),
        mesh=mesh,
        scratch_types=[
            pltpu.VMEM((nchunk, _CHUNK), jnp.int32),
            pltpu.VMEM((_NBUF, _CHUNK, _D), jnp.float32),
            pltpu.VMEM((_NBUF, _DT, 8, _CHUNK), jnp.float32),
            pltpu.SemaphoreType.DMA,
            pltpu.SemaphoreType.DMA,
        ],
        compiler_params=pltpu.CompilerParams(
            use_tc_tiling_on_sc=False, needs_layout_passes=False),
    )(xt, table)
    return out5.transpose(2, 4, 0, 1, 3).reshape(bsz, n_s, _D)


# speedup vs baseline: 1.3218x; 1.3218x over previous
import jax
import jax.numpy as jnp
from jax import lax
from jax.experimental import pallas as pl
from jax.experimental.pallas import tpu as pltpu
from jax.experimental.pallas import tpu_sc as plsc

_NC = 2
_NS = 16
_NW = _NC * _NS
_CHUNK = 128
_D = 64
_DT = _D // 8
_NBUF = 4
_AHEAD = 2

_V = 1000000
_NT_FULL = _V // _CHUNK
_TAIL = _V - _NT_FULL * _CHUNK
_BASE_T = _NT_FULL // _NW
_EXTRA = _NT_FULL - _BASE_T * _NW


def _tr_body(tt_hbm, tail_hbm, out_hbm,
             vin0, vin1, vin2, vin3, vout0, vout1, vout2, vout3,
             tail_v, gsem, ssem):
    vins = [vin0, vin1, vin2, vin3]
    vouts = [vout0, vout1, vout2, vout3]
    wid = lax.axis_index("s") * _NC + lax.axis_index("c")
    iota = lax.broadcasted_iota(jnp.int32, (16,), 0)
    t0 = wid * _BASE_T

    def g_desc(i, b):
        return pltpu.make_async_copy(
            tt_hbm.at[:, pl.ds(i * _CHUNK, _CHUNK)], vins[b], gsem)

    def s_desc(i, b):
        return pltpu.make_async_copy(
            vouts[b], out_hbm.at[pl.ds(i * _CHUNK * _D, _CHUNK * _D)], ssem)

    @pl.when(wid == _NW - 1)
    def _():
        pltpu.sync_copy(tail_hbm, tail_v)
        pltpu.sync_copy(tail_v, out_hbm.at[pl.ds(_NT_FULL * _CHUNK * _D,
                                                 _TAIL * _D)])

    def transpose_tile(b):
        vin_b = vins[b]
        vout_b = vouts[b]

        @plsc.parallel_loop(0, _CHUNK, unroll=8)
        def _(c):
            col = jnp.full((16,), c, jnp.int32)
            for m in range(4):
                v = plsc.load_gather(vin_b, [iota + 16 * m, col])
                vout_b[pl.ds(c * _D + 16 * m, 16)] = v

    for b in range(_AHEAD):
        g_desc(t0 + b, b).start()

    ngrp = _BASE_T // _NBUF

    def group(g, carry):
        for b in range(_NBUF):
            i = t0 + g * _NBUF + b

            @pl.when(g * _NBUF + b + _AHEAD < _BASE_T)
            def _():
                g_desc(i + _AHEAD, (b + _AHEAD) % _NBUF).start()

            g_desc(i, b).wait()

            @pl.when(g >= 1)
            def _():
                s_desc(i - _NBUF, b).wait()

            transpose_tile(b)
            s_desc(i, b).start()
        return carry

    lax.fori_loop(0, ngrp, group, 0)
    for b in range(_NBUF):
        s_desc(t0 + _BASE_T - _NBUF + b, b).wait()

    @pl.when(wid < _EXTRA)
    def _():
        i = _NW * _BASE_T + wid
        g_desc(i, 0).start()
        g_desc(i, 0).wait()
        transpose_tile(0)
        s_desc(i, 0).start()
        s_desc(i, 0).wait()


def _gather_body(x_hbm, table_hbm, out_hbm, idx_v, rows_v, tile_v, gsem, ssem):
    nchunk = x_hbm.shape[1]
    n_s = out_hbm.shape[0]
    wid = lax.axis_index("s") * _NC + lax.axis_index("c")
    pltpu.sync_copy(x_hbm.at[wid], idx_v)
    iota = lax.broadcasted_iota(jnp.int32, (16,), 0)

    def g_desc(j, b):
        return pltpu.make_async_copy(
            table_hbm.at[idx_v.at[j]], rows_v.at[b], gsem)

    def s_desc(j, b):
        t = j // n_s
        s = j - t * n_s
        return pltpu.make_async_copy(
            tile_v.at[b], out_hbm.at[s, :, wid * (nchunk // n_s) + t], ssem)

    for j in range(_AHEAD):
        g_desc(j, j % _NBUF).start()

    def transpose_chunk(b):
        rows_b = rows_v.at[b]
        tile_b = tile_v.at[b]

        @plsc.parallel_loop(0, _D, unroll=8)
        def _(d):
            dt = d // 8
            ds = d - dt * 8
            col = jnp.full((16,), d, jnp.int32)
            for k in range(8):
                v = plsc.load_gather(rows_b, [iota + 16 * k, col])
                tile_b[dt, ds, pl.ds(16 * k, 16)] = v

    def group(g, carry):
        for b in range(_NBUF):
            j = g * _NBUF + b

            @pl.when(j + _AHEAD < nchunk)
            def _():
                g_desc(j + _AHEAD, (b + _AHEAD) % _NBUF).start()

            g_desc(j, b).wait()

            @pl.when(g >= 1)
            def _():
                s_desc(j - _NBUF, b).wait()

            transpose_chunk(b)
            s_desc(j, b).start()
        return carry

    lax.fori_loop(0, nchunk // _NBUF, group, 0)
    for j in range(nchunk - _NBUF, nchunk):
        s_desc(j, j % _NBUF).wait()


def kernel(x, table):
    bsz, n_s = x.shape
    nbt = bsz // _CHUNK
    tpw = nbt // _NW
    nchunk = tpw * n_s
    mesh = plsc.VectorSubcoreMesh(core_axis_name="c", subcore_axis_name="s")

    tt = table.T
    tail = table[_NT_FULL * _CHUNK:, :].reshape(-1)
    tlin = pl.kernel(
        _tr_body,
        out_type=jax.ShapeDtypeStruct((_V * _D,), jnp.float32),
        mesh=mesh,
        scratch_types=(
            [pltpu.VMEM((_D, _CHUNK), jnp.float32)] * _NBUF
            + [pltpu.VMEM((_CHUNK * _D,), jnp.float32)] * _NBUF
            + [
                pltpu.VMEM((_TAIL * _D,), jnp.float32),
                pltpu.SemaphoreType.DMA,
                pltpu.SemaphoreType.DMA,
            ]
        ),
        compiler_params=pltpu.CompilerParams(
            use_tc_tiling_on_sc=True, needs_layout_passes=False),
    )(tt, tail)
    tbl2 = tlin.reshape(_V, _D)

    xt = (x.reshape(_NW, tpw, _CHUNK, n_s)
          .transpose(0, 1, 3, 2)
          .reshape(_NW, nchunk, _CHUNK)
          .astype(jnp.int32))
    out5 = pl.kernel(
        _gather_body,
        out_type=jax.ShapeDtypeStruct((n_s, _DT, nbt, 8, _CHUNK), jnp.float32),
        mesh=mesh,
        scratch_types=[
            pltpu.VMEM((nchunk, _CHUNK), jnp.int32),
            pltpu.VMEM((_NBUF, _CHUNK, _D), jnp.float32),
            pltpu.VMEM((_NBUF, _DT, 8, _CHUNK), jnp.float32),
            pltpu.SemaphoreType.DMA,
            pltpu.SemaphoreType.DMA,
        ],
        compiler_params=pltpu.CompilerParams(
            use_tc_tiling_on_sc=False, needs_layout_passes=False),
    )(xt, tbl2)
    return out5.transpose(2, 4, 0, 1, 3).reshape(bsz, n_s, _D)


# speedup vs baseline: 3.7780x; 2.8582x over previous
import jax
import jax.numpy as jnp
from jax import lax
from jax.experimental import pallas as pl
from jax.experimental.pallas import tpu as pltpu
from jax.experimental.pallas import tpu_sc as plsc

_NC = 2
_NS = 16
_NW = _NC * _NS
_CHUNK = 128
_D = 64
_DT = _D // 8
_NBUF = 4
_AHEAD = 2

_V = 1000000
_NT_FULL = _V // _CHUNK
_TAIL = _V - _NT_FULL * _CHUNK
_BASE_T = _NT_FULL // _NW
_EXTRA = _NT_FULL - _BASE_T * _NW


def _tr_body(tt_hbm, tail_hbm, out_hbm,
             vin0, vin1, vin2, vin3, vout0, vout1, vout2, vout3,
             tail_v, gsem, ssem):
    vins = [vin0, vin1, vin2, vin3]
    vouts = [vout0, vout1, vout2, vout3]
    wid = lax.axis_index("s") * _NC + lax.axis_index("c")
    iota = lax.broadcasted_iota(jnp.int32, (16,), 0)
    t0 = wid * _BASE_T

    def g_desc(i, b):
        return pltpu.make_async_copy(
            tt_hbm.at[:, pl.ds(i * _CHUNK, _CHUNK)], vins[b], gsem)

    def s_desc(i, b):
        return pltpu.make_async_copy(
            vouts[b], out_hbm.at[pl.ds(i * _CHUNK * _D, _CHUNK * _D)], ssem)

    @pl.when(wid == _NW - 1)
    def _():
        pltpu.sync_copy(tail_hbm, tail_v)
        pltpu.sync_copy(tail_v, out_hbm.at[pl.ds(_NT_FULL * _CHUNK * _D,
                                                 _TAIL * _D)])

    def transpose_tile(b):
        vin_b = vins[b]
        vout_b = vouts[b]

        @plsc.parallel_loop(0, 32, unroll=2)
        def _(blk):
            m = blk // 8
            k = blk - 8 * m
            colv = iota + 16 * k
            flatbase = colv * _D + 16 * m
            for j in range(16):
                perm = (iota + j) & 15
                v = plsc.load_gather(vin_b, [perm + 16 * m, colv])
                plsc.store_scatter(vout_b, [flatbase + perm], v)

    for b in range(_AHEAD):
        g_desc(t0 + b, b).start()

    ngrp = _BASE_T // _NBUF

    def group(g, carry):
        for b in range(_NBUF):
            i = t0 + g * _NBUF + b

            @pl.when(g * _NBUF + b + _AHEAD < _BASE_T)
            def _():
                g_desc(i + _AHEAD, (b + _AHEAD) % _NBUF).start()

            g_desc(i, b).wait()

            @pl.when(g >= 1)
            def _():
                s_desc(i - _NBUF, b).wait()

            transpose_tile(b)
            s_desc(i, b).start()
        return carry

    lax.fori_loop(0, ngrp, group, 0)
    for b in range(_NBUF):
        s_desc(t0 + _BASE_T - _NBUF + b, b).wait()

    @pl.when(wid < _EXTRA)
    def _():
        i = _NW * _BASE_T + wid
        g_desc(i, 0).start()
        g_desc(i, 0).wait()
        transpose_tile(0)
        s_desc(i, 0).start()
        s_desc(i, 0).wait()


def _gather_body(x_hbm, table_hbm, out_hbm, idx_v, rows_v, tile_v, gsem, ssem):
    nchunk = x_hbm.shape[1]
    n_s = out_hbm.shape[0]
    wid = lax.axis_index("s") * _NC + lax.axis_index("c")
    pltpu.sync_copy(x_hbm.at[wid], idx_v)
    iota = lax.broadcasted_iota(jnp.int32, (16,), 0)

    def g_desc(j, b):
        return pltpu.make_async_copy(
            table_hbm.at[idx_v.at[j]], rows_v.at[b], gsem)

    def s_desc(j, b):
        t = j // n_s
        s = j - t * n_s
        return pltpu.make_async_copy(
            tile_v.at[b], out_hbm.at[s, :, wid * (nchunk // n_s) + t], ssem)

    for j in range(_AHEAD):
        g_desc(j, j % _NBUF).start()

    def transpose_chunk(b):
        rows_b = rows_v.at[b]
        tile_b = tile_v.at[b]

        @plsc.parallel_loop(0, 32, unroll=2)
        def _(blk):
            k = blk // 4
            m = blk - 4 * k
            colv = iota + 16 * m
            dtv = colv >> 3
            innerbase = (colv & 7) * _CHUNK + 16 * k
            for j in range(16):
                perm = (iota + j) & 15
                v = plsc.load_gather(rows_b, [perm + 16 * k, colv])
                plsc.store_scatter(tile_b, [dtv, innerbase + perm], v)

    def group(g, carry):
        for b in range(_NBUF):
            j = g * _NBUF + b

            @pl.when(j + _AHEAD < nchunk)
            def _():
                g_desc(j + _AHEAD, (b + _AHEAD) % _NBUF).start()

            g_desc(j, b).wait()

            @pl.when(g >= 1)
            def _():
                s_desc(j - _NBUF, b).wait()

            transpose_chunk(b)
            s_desc(j, b).start()
        return carry

    lax.fori_loop(0, nchunk // _NBUF, group, 0)
    for j in range(nchunk - _NBUF, nchunk):
        s_desc(j, j % _NBUF).wait()


def kernel(x, table):
    bsz, n_s = x.shape
    nbt = bsz // _CHUNK
    tpw = nbt // _NW
    nchunk = tpw * n_s
    mesh = plsc.VectorSubcoreMesh(core_axis_name="c", subcore_axis_name="s")

    tt = table.T
    tail = table[_NT_FULL * _CHUNK:, :].reshape(-1)
    tlin = pl.kernel(
        _tr_body,
        out_type=jax.ShapeDtypeStruct((_V * _D,), jnp.float32),
        mesh=mesh,
        scratch_types=(
            [pltpu.VMEM((_D, _CHUNK), jnp.float32)] * _NBUF
            + [pltpu.VMEM((_CHUNK * _D,), jnp.float32)] * _NBUF
            + [
                pltpu.VMEM((_TAIL * _D,), jnp.float32),
                pltpu.SemaphoreType.DMA,
                pltpu.SemaphoreType.DMA,
            ]
        ),
        compiler_params=pltpu.CompilerParams(
            use_tc_tiling_on_sc=True, needs_layout_passes=False),
    )(tt, tail)
    tbl2 = tlin.reshape(_V, _D)

    xt = (x.reshape(_NW, tpw, _CHUNK, n_s)
          .transpose(0, 1, 3, 2)
          .reshape(_NW, nchunk, _CHUNK)
          .astype(jnp.int32))
    out5 = pl.kernel(
        _gather_body,
        out_type=jax.ShapeDtypeStruct((n_s, _DT, nbt, 8 * _CHUNK), jnp.float32),
        mesh=mesh,
        scratch_types=[
            pltpu.VMEM((nchunk, _CHUNK), jnp.int32),
            pltpu.VMEM((_NBUF, _CHUNK, _D), jnp.float32),
            pltpu.VMEM((_NBUF, _DT, 8 * _CHUNK), jnp.float32),
            pltpu.SemaphoreType.DMA,
            pltpu.SemaphoreType.DMA,
        ],
        compiler_params=pltpu.CompilerParams(
            use_tc_tiling_on_sc=False, needs_layout_passes=False),
    )(xt, tbl2)
    return (out5.reshape(n_s, _DT, nbt, 8, _CHUNK)
            .transpose(2, 4, 0, 1, 3)
            .reshape(bsz, n_s, _D))
